# baseline (device time: 100408 ns/iter reference)
import functools

import jax
import jax.numpy as jnp
from jax import lax
from jax.experimental import pallas as pl
from jax.experimental.pallas import tpu as pltpu

N_X = 2
S_LOCAL = 1024
S_GLOBAL = N_X * S_LOCAL
H = 16
D = 128
SCALE = D ** -0.5

HALF = S_LOCAL // 2
N_CHUNK = 4
CHUNK = HALF // N_CHUNK


def _body(q_ref, k_ref, v_ref, o_ref,
          kf_ref, vf_ref, qt_ref, oacc_ref, oh_ref, stg_ref,
          x_send, x_recv, y_send, y_recv, stg_sems, o_sems):
    my_x = lax.axis_index("x")
    my_y = lax.axis_index("y")
    my_z = lax.axis_index("z")
    xpeer = (1 - my_x, my_y, my_z)
    parity = lax.rem(my_y, 2)
    ypart = (my_x, my_y + 1 - 2 * parity, my_z)

    barrier = pltpu.get_barrier_semaphore()
    for nbr in (xpeer, ypart):
        pl.semaphore_signal(
            barrier, inc=1, device_id=nbr, device_id_type=pl.DeviceIdType.MESH
        )
    pl.semaphore_wait(barrier, 2)

    my_base = my_x * S_LOCAL
    peer_base = (1 - my_x) * S_LOCAL
    h_mine = parity * HALF
    h_other = HALF - h_mine

    dsts = (kf_ref, vf_ref)

    def convert(src_hbm, store):
        cps = {}
        for h in range(H):
            if h == 0:
                cps[0] = pltpu.make_async_copy(
                    src_hbm.at[:, 0, :], stg_ref.at[0], stg_sems.at[0]
                )
                cps[0].start()
            if h + 1 < H:
                cps[h + 1] = pltpu.make_async_copy(
                    src_hbm.at[:, h + 1, :],
                    stg_ref.at[(h + 1) % 2],
                    stg_sems.at[(h + 1) % 2],
                )
                cps[h + 1].start()
            cps[h].wait()
            store(h, stg_ref[h % 2].astype(jnp.bfloat16))

    def _store_kv(dst):
        def store(h, x):
            dst[h, pl.ds(my_base, S_LOCAL)] = x
        return store

    convert(k_ref, _store_kv(kf_ref))
    convert(v_ref, _store_kv(vf_ref))

    x_rdmas = {}
    for c in range(N_CHUNK):
        for t in range(2):
            off = h_mine + c * CHUNK
            rdma = pltpu.make_async_remote_copy(
                src_ref=dsts[t].at[:, pl.ds(my_base + off, CHUNK), :],
                dst_ref=dsts[t].at[:, pl.ds(my_base + off, CHUNK), :],
                send_sem=x_send.at[t, c],
                recv_sem=x_recv.at[t, c],
                device_id=xpeer,
                device_id_type=pl.DeviceIdType.MESH,
            )
            rdma.start()
            x_rdmas[t, c] = rdma

    def _store_q(h, x):
        qt_ref[h] = x

    convert(q_ref, _store_q)

    l_vals = [None] * H

    def accumulate(start, length, first=False):
        for h in range(H):
            qh = qt_ref[h]
            kh = kf_ref[h, pl.ds(start, length), :]
            vh = vf_ref[h, pl.ds(start, length), :]
            s = lax.dot_general(
                qh, kh, (((1,), (1,)), ((), ())),
                preferred_element_type=jnp.float32,
            ) * SCALE
            p = jnp.exp(s)
            l = jnp.sum(p, axis=-1, keepdims=True)
            o = lax.dot_general(
                p.astype(jnp.bfloat16), vh, (((1,), (0,)), ((), ())),
                preferred_element_type=jnp.float32,
            )
            if first:
                oacc_ref[h] = o
                l_vals[h] = l
            else:
                oacc_ref[h] = oacc_ref[h] + o
                l_vals[h] = l_vals[h] + l

    y_rdmas = []
    for c in range(N_CHUNK):
        accumulate(my_base + c * CHUNK * 2, CHUNK * 2, first=(c == 0))
        off = peer_base + h_mine + c * CHUNK
        for t in range(2):
            x_rdmas[t, c].wait_recv()
            fwd = pltpu.make_async_remote_copy(
                src_ref=dsts[t].at[:, pl.ds(off, CHUNK), :],
                dst_ref=dsts[t].at[:, pl.ds(off, CHUNK), :],
                send_sem=y_send.at[t, c],
                recv_sem=y_recv.at[t, c],
                device_id=ypart,
                device_id_type=pl.DeviceIdType.MESH,
            )
            fwd.start()
            y_rdmas.append(fwd)
        accumulate(off, CHUNK)

    for c in range(N_CHUNK):
        off = peer_base + h_other + c * CHUNK
        for t in range(2):
            recv = pltpu.make_async_remote_copy(
                src_ref=dsts[t].at[:, pl.ds(off, CHUNK), :],
                dst_ref=dsts[t].at[:, pl.ds(off, CHUNK), :],
                send_sem=y_send.at[t, c],
                recv_sem=y_recv.at[t, c],
                device_id=ypart,
                device_id_type=pl.DeviceIdType.MESH,
            )
            recv.wait_recv()
        accumulate(off, CHUNK)

    ocps = [None, None]
    for h in range(H):
        slot = h % 2
        if ocps[slot] is not None:
            ocps[slot].wait()
        oh_ref[slot] = oacc_ref[h] / l_vals[h]
        cp = pltpu.make_async_copy(
            oh_ref.at[slot], o_ref.at[:, h, :], o_sems.at[slot]
        )
        cp.start()
        ocps[slot] = cp
    for cp in ocps:
        cp.wait()

    for rdma in x_rdmas.values():
        rdma.wait_send()
    for fwd in y_rdmas:
        fwd.wait_send()

    @functools.partial(pl.run_scoped, exit_sem=pltpu.SemaphoreType.REGULAR)
    def _(exit_sem):
        for nbr in (xpeer, ypart):
            pl.semaphore_signal(
                exit_sem, inc=1, device_id=nbr,
                device_id_type=pl.DeviceIdType.MESH,
            )
        pl.semaphore_wait(exit_sem, 2)


def kernel(Q, K, V):
    q = Q.reshape(S_LOCAL, H, D)
    k = K.reshape(S_LOCAL, H, D)
    v = V.reshape(S_LOCAL, H, D)

    out = pl.pallas_call(
        _body,
        out_shape=jax.ShapeDtypeStruct((S_LOCAL, H, D), jnp.float32),
        in_specs=[pl.BlockSpec(memory_space=pl.MemorySpace.ANY)] * 3,
        out_specs=pl.BlockSpec(memory_space=pl.MemorySpace.ANY),
        scratch_shapes=[
            pltpu.VMEM((H, S_GLOBAL, D), jnp.bfloat16),
            pltpu.VMEM((H, S_GLOBAL, D), jnp.bfloat16),
            pltpu.VMEM((H, S_LOCAL, D), jnp.bfloat16),
            pltpu.VMEM((H, S_LOCAL, D), jnp.float32),
            pltpu.VMEM((2, S_LOCAL, D), jnp.float32),
            pltpu.VMEM((2, S_LOCAL, D), jnp.float32),
            pltpu.SemaphoreType.DMA((2, N_CHUNK)),
            pltpu.SemaphoreType.DMA((2, N_CHUNK)),
            pltpu.SemaphoreType.DMA((2, N_CHUNK)),
            pltpu.SemaphoreType.DMA((2, N_CHUNK)),
            pltpu.SemaphoreType.DMA((2,)),
            pltpu.SemaphoreType.DMA((2,)),
        ],
        compiler_params=pltpu.CompilerParams(
            collective_id=0, vmem_limit_bytes=64 * 1024 * 1024
        ),
    )(q, k, v)
    return out.reshape(1, S_LOCAL, H, D)


# device time: 97543 ns/iter; 1.0294x vs baseline; 1.0294x over previous
import functools

import jax
import jax.numpy as jnp
from jax import lax
from jax.experimental import pallas as pl
from jax.experimental.pallas import tpu as pltpu

N_X = 2
S_LOCAL = 1024
S_GLOBAL = N_X * S_LOCAL
H = 16
D = 128
SCALE = D ** -0.5

HALF = S_LOCAL // 2
N_CHUNK = 4
CHUNK = HALF // N_CHUNK


def _body(q_ref, k_ref, v_ref, o_ref,
          kf_ref, vf_ref, qt_ref, oacc_ref, oh_ref,
          x_send, x_recv, y_send, y_recv, cp_sems, q_sems, o_sems):
    my_x = lax.axis_index("x")
    my_y = lax.axis_index("y")
    my_z = lax.axis_index("z")
    xpeer = (1 - my_x, my_y, my_z)
    parity = lax.rem(my_y, 2)
    ypart = (my_x, my_y + 1 - 2 * parity, my_z)

    barrier = pltpu.get_barrier_semaphore()
    for nbr in (xpeer, ypart):
        pl.semaphore_signal(
            barrier, inc=1, device_id=nbr, device_id_type=pl.DeviceIdType.MESH
        )
    pl.semaphore_wait(barrier, 2)

    my_base = my_x * S_LOCAL
    peer_base = (1 - my_x) * S_LOCAL
    h_mine = parity * HALF
    h_other = HALF - h_mine

    srcs = (k_ref, v_ref)
    dsts = (kf_ref, vf_ref)

    cps = []
    for t in range(2):
        cp = pltpu.make_async_copy(
            srcs[t], dsts[t].at[:, pl.ds(my_base, S_LOCAL), :], cp_sems.at[t]
        )
        cp.start()
        cps.append(cp)
    cp_q = pltpu.make_async_copy(q_ref, qt_ref, q_sems.at[0])
    cp_q.start()

    x_rdmas = {}
    for c in range(N_CHUNK):
        for t in range(2):
            off = h_mine + c * CHUNK
            rdma = pltpu.make_async_remote_copy(
                src_ref=srcs[t].at[:, pl.ds(off, CHUNK), :],
                dst_ref=dsts[t].at[:, pl.ds(my_base + off, CHUNK), :],
                send_sem=x_send.at[t, c],
                recv_sem=x_recv.at[t, c],
                device_id=xpeer,
                device_id_type=pl.DeviceIdType.MESH,
            )
            rdma.start()
            x_rdmas[t, c] = rdma

    for cp in cps:
        cp.wait()
    cp_q.wait()

    l_vals = [None] * H

    def head_update(h, start, length):
        qh = qt_ref[h]
        kh = kf_ref[h, pl.ds(start, length), :]
        vh = vf_ref[h, pl.ds(start, length), :]
        s = lax.dot_general(
            qh, kh, (((1,), (1,)), ((), ())),
            preferred_element_type=jnp.float32,
        ) * SCALE
        p = jnp.exp(s)
        l = jnp.sum(p, axis=-1, keepdims=True)
        o = lax.dot_general(
            p.astype(jnp.bfloat16), vh, (((1,), (0,)), ((), ())),
            preferred_element_type=jnp.float32,
        )
        return o, l

    def accumulate(start, length, first=False):
        for h in range(H):
            o, l = head_update(h, start, length)
            if first:
                oacc_ref[h] = o
                l_vals[h] = l
            else:
                oacc_ref[h] = oacc_ref[h] + o
                l_vals[h] = l_vals[h] + l

    y_rdmas = []
    for c in range(N_CHUNK):
        accumulate(my_base + c * CHUNK * 2, CHUNK * 2, first=(c == 0))
        off = peer_base + h_mine + c * CHUNK
        for t in range(2):
            x_rdmas[t, c].wait_recv()
            fwd = pltpu.make_async_remote_copy(
                src_ref=dsts[t].at[:, pl.ds(off, CHUNK), :],
                dst_ref=dsts[t].at[:, pl.ds(off, CHUNK), :],
                send_sem=y_send.at[t, c],
                recv_sem=y_recv.at[t, c],
                device_id=ypart,
                device_id_type=pl.DeviceIdType.MESH,
            )
            fwd.start()
            y_rdmas.append(fwd)
        accumulate(off, CHUNK)

    ocps = [None, None]
    for c in range(N_CHUNK):
        off = peer_base + h_other + c * CHUNK
        for t in range(2):
            recv = pltpu.make_async_remote_copy(
                src_ref=dsts[t].at[:, pl.ds(off, CHUNK), :],
                dst_ref=dsts[t].at[:, pl.ds(off, CHUNK), :],
                send_sem=y_send.at[t, c],
                recv_sem=y_recv.at[t, c],
                device_id=ypart,
                device_id_type=pl.DeviceIdType.MESH,
            )
            recv.wait_recv()
        if c + 1 < N_CHUNK:
            accumulate(off, CHUNK)
        else:
            for h in range(H):
                o, l = head_update(h, off, CHUNK)
                slot = h % 2
                if ocps[slot] is not None:
                    ocps[slot].wait()
                oh_ref[slot] = (oacc_ref[h] + o) / (l_vals[h] + l)
                cp = pltpu.make_async_copy(
                    oh_ref.at[slot], o_ref.at[:, h, :], o_sems.at[slot]
                )
                cp.start()
                ocps[slot] = cp
    for cp in ocps:
        cp.wait()

    for rdma in x_rdmas.values():
        rdma.wait_send()
    for fwd in y_rdmas:
        fwd.wait_send()

    @functools.partial(pl.run_scoped, exit_sem=pltpu.SemaphoreType.REGULAR)
    def _(exit_sem):
        for nbr in (xpeer, ypart):
            pl.semaphore_signal(
                exit_sem, inc=1, device_id=nbr,
                device_id_type=pl.DeviceIdType.MESH,
            )
        pl.semaphore_wait(exit_sem, 2)


def kernel(Q, K, V):
    q = Q.reshape(S_LOCAL, H, D).astype(jnp.bfloat16).transpose(1, 0, 2)
    k = K.reshape(S_LOCAL, H, D).astype(jnp.bfloat16).transpose(1, 0, 2)
    v = V.reshape(S_LOCAL, H, D).astype(jnp.bfloat16).transpose(1, 0, 2)

    out = pl.pallas_call(
        _body,
        out_shape=jax.ShapeDtypeStruct((S_LOCAL, H, D), jnp.float32),
        in_specs=[pl.BlockSpec(memory_space=pl.MemorySpace.ANY)] * 3,
        out_specs=pl.BlockSpec(memory_space=pl.MemorySpace.ANY),
        scratch_shapes=[
            pltpu.VMEM((H, S_GLOBAL, D), jnp.bfloat16),
            pltpu.VMEM((H, S_GLOBAL, D), jnp.bfloat16),
            pltpu.VMEM((H, S_LOCAL, D), jnp.bfloat16),
            pltpu.VMEM((H, S_LOCAL, D), jnp.float32),
            pltpu.VMEM((2, S_LOCAL, D), jnp.float32),
            pltpu.SemaphoreType.DMA((2, N_CHUNK)),
            pltpu.SemaphoreType.DMA((2, N_CHUNK)),
            pltpu.SemaphoreType.DMA((2, N_CHUNK)),
            pltpu.SemaphoreType.DMA((2, N_CHUNK)),
            pltpu.SemaphoreType.DMA((2,)),
            pltpu.SemaphoreType.DMA((1,)),
            pltpu.SemaphoreType.DMA((2,)),
        ],
        compiler_params=pltpu.CompilerParams(
            collective_id=0, vmem_limit_bytes=64 * 1024 * 1024
        ),
    )(q, k, v)
    return out.reshape(1, S_LOCAL, H, D)
